# baseline (speedup 1.0000x reference)
import functools

import jax
import jax.numpy as jnp
from jax import lax
from jax.experimental import pallas as pl
from jax.experimental.pallas import tpu as pltpu
from jax.experimental.pallas import tpu_sc as plsc

_B_OUT = 4096


@functools.lru_cache(maxsize=None)
def _build(t_rows: int, dim: int):
    lanes = 16
    mesh = plsc.VectorSubcoreMesh(core_axis_name="c", subcore_axis_name="s")
    nc, ns = mesh.num_cores, mesh.num_subcores
    nw = nc * ns
    assert _B_OUT % nw == 0
    rows_per_w = _B_OUT // nw
    row_words = t_rows * dim

    @functools.partial(
        pl.kernel,
        mesh=mesh,
        out_type=jax.ShapeDtypeStruct((_B_OUT, row_words), jnp.float32),
        scratch_types=[
            pltpu.VMEM((row_words,), jnp.float32),
            pltpu.VMEM((dim,), jnp.float32),
            pltpu.SemaphoreType.DMA,
        ],
    )
    def k(temb_hbm, bias_hbm, out_hbm, tab_v, bias_v, sem):
        wid = lax.axis_index("s") * nc + lax.axis_index("c")
        pltpu.sync_copy(temb_hbm, tab_v)
        pltpu.sync_copy(bias_hbm, bias_v)

        def add_row(j, carry):
            for kk in range(dim // lanes):
                sl = pl.ds(j * dim + kk * lanes, lanes)
                tab_v[sl] = tab_v[sl] + bias_v[pl.ds(kk * lanes, lanes)]
            return carry

        lax.fori_loop(0, t_rows, add_row, 0)

        base = wid * rows_per_w

        def issue(r, carry):
            pltpu.async_copy(tab_v, out_hbm.at[base + r], sem)
            return carry

        lax.fori_loop(0, rows_per_w, issue, 0)

        def drain(r, carry):
            pltpu.make_async_copy(tab_v, out_hbm.at[base + r], sem).wait()
            return carry

        lax.fori_loop(0, rows_per_w, drain, 0)

    return k


def kernel(B, T, temb_weight, type_bias):
    t_rows, dim = temb_weight.shape
    temb_flat = temb_weight.reshape(t_rows * dim)
    bias_flat = type_bias.reshape(dim)
    out = _build(t_rows, dim)(temb_flat, bias_flat)
    return out.reshape(_B_OUT, t_rows, 1, dim)
